# baseline (speedup 1.0000x reference)
import functools

import jax
import jax.numpy as jnp
from jax import lax
from jax.experimental import pallas as pl
from jax.experimental.pallas import tpu as pltpu
from jax.experimental.pallas import tpu_sc as plsc

L = 16
NC = 2
NS = 16
NW = NC * NS

D = 128
G = D // L
NBUF = 4
RU = 5


@jax.jit
def _sc_permute(x, perm):
    B, S, _ = x.shape
    batches_per_w = B // NW
    nt = batches_per_w // NBUF
    mesh = plsc.VectorSubcoreMesh(core_axis_name="c", subcore_axis_name="s")

    @functools.partial(
        pl.kernel,
        mesh=mesh,
        compiler_params=pltpu.CompilerParams(needs_layout_passes=False),
        out_type=jax.ShapeDtypeStruct((B, S, D), jnp.float32),
        scratch_types=(
            [pltpu.VMEM((D,), jnp.int32)]
            + [pltpu.VMEM((S, D), jnp.float32) for _ in range(2 * NBUF)]
            + [pltpu.SemaphoreType.DMA for _ in range(2 * NBUF)]
        ),
    )
    def k(x_hbm, perm_hbm, out_hbm, perm_v,
          i0, i1, i2, i3, o0, o1, o2, o3,
          si0, si1, si2, si3, so0, so1, so2, so3):
        ins = (i0, i1, i2, i3)
        outs = (o0, o1, o2, o3)
        sins = (si0, si1, si2, si3)
        souts = (so0, so1, so2, so3)

        wid = lax.axis_index("s") * NC + lax.axis_index("c")
        bbase = wid * batches_per_w
        pltpu.sync_copy(perm_hbm, perm_v)
        cols = tuple(perm_v[pl.ds(g * L, L)] for g in range(G))

        def cp_in(t, b):
            return pltpu.make_async_copy(x_hbm.at[bbase + t], ins[b], sins[b])

        def cp_out(t, b):
            return pltpu.make_async_copy(outs[b], out_hbm.at[bbase + t], souts[b])

        for b in range(NBUF):
            cp_in(b, b).start()

        def permute(inb, oub):
            def row_body(r5, c):
                for u in range(RU):
                    r = r5 * RU + u
                    rv = jnp.full((L,), r, dtype=jnp.int32)
                    for g in range(G):
                        oub[r, pl.ds(g * L, L)] = plsc.load_gather(
                            inb, [rv, cols[g]])
                return c

            lax.fori_loop(0, S // RU, row_body, 0)

        def outer(t4, c):
            for b in range(NBUF):
                t = t4 * NBUF + b
                cp_in(t, b).wait()

                @pl.when(t4 > 0)
                def _():
                    cp_out(t - NBUF, b).wait()

                permute(ins[b], outs[b])
                cp_out(t, b).start()

                @pl.when(t4 + 1 < nt)
                def _():
                    cp_in(t + NBUF, b).start()

            return c

        lax.fori_loop(0, nt, outer, 0)
        for b in range(NBUF):
            cp_out((nt - 1) * NBUF + b, b).wait()

    return k(x, perm)


def kernel(input, permutation):
    return _sc_permute(input, permutation.astype(jnp.int32))


# speedup vs baseline: 1.5730x; 1.5730x over previous
import functools

import jax
import jax.numpy as jnp
from jax import lax
from jax.experimental import pallas as pl
from jax.experimental.pallas import tpu as pltpu
from jax.experimental.pallas import tpu_sc as plsc

L = 16
NC = 2
NS = 16
NW = NC * NS

D = 128
G = D // L
NBUF = 4
RU = 5


@jax.jit
def _sc_permute(x, perm):
    B, S, _ = x.shape
    batches_per_w = B // NW
    nt = batches_per_w // NBUF
    mesh = plsc.VectorSubcoreMesh(core_axis_name="c", subcore_axis_name="s")

    @functools.partial(
        pl.kernel,
        mesh=mesh,
        compiler_params=pltpu.CompilerParams(needs_layout_passes=False),
        out_type=jax.ShapeDtypeStruct((B, S, D), jnp.float32),
        scratch_types=(
            [pltpu.VMEM((D,), jnp.int32)]
            + [pltpu.VMEM((S, D), jnp.float32) for _ in range(2 * NBUF)]
            + [pltpu.SemaphoreType.DMA for _ in range(2 * NBUF)]
        ),
    )
    def k(x_hbm, perm_hbm, out_hbm, perm_v,
          i0, i1, i2, i3, o0, o1, o2, o3,
          si0, si1, si2, si3, so0, so1, so2, so3):
        ins = (i0, i1, i2, i3)
        outs = (o0, o1, o2, o3)
        sins = (si0, si1, si2, si3)
        souts = (so0, so1, so2, so3)

        wid = lax.axis_index("s") * NC + lax.axis_index("c")
        bbase = wid * batches_per_w
        pltpu.sync_copy(perm_hbm, perm_v)
        cols = tuple(perm_v[pl.ds(g * L, L)] for g in range(G))

        def cp_in(t, b):
            return pltpu.make_async_copy(x_hbm.at[bbase + t], ins[b], sins[b])

        def cp_out(t, b):
            return pltpu.make_async_copy(outs[b], out_hbm.at[bbase + t], souts[b])

        for b in range(NBUF):
            cp_in(b, b).start()

        def permute(inb, oub):
            def row_body(r5, c):
                vals = []
                for u in range(RU):
                    rv = jnp.full((L,), r5 * RU + u, dtype=jnp.int32)
                    vals.append(
                        [plsc.load_gather(inb, [rv, cols[g]]) for g in range(G)])
                for u in range(RU):
                    r = r5 * RU + u
                    for g in range(G):
                        oub[r, pl.ds(g * L, L)] = vals[u][g]
                return c

            lax.fori_loop(0, S // RU, row_body, 0)

        def outer(t4, c):
            for b in range(NBUF):
                t = t4 * NBUF + b
                cp_in(t, b).wait()

                @pl.when(t4 > 0)
                def _():
                    cp_out(t - NBUF, b).wait()

                permute(ins[b], outs[b])
                cp_out(t, b).start()

                @pl.when(t4 + 1 < nt)
                def _():
                    cp_in(t + NBUF, b).start()

            return c

        lax.fori_loop(0, nt, outer, 0)
        for b in range(NBUF):
            cp_out((nt - 1) * NBUF + b, b).wait()

    return k(x, perm)


def kernel(input, permutation):
    return _sc_permute(input, permutation.astype(jnp.int32))
